# baseline (speedup 1.0000x reference)
import jax
import jax.numpy as jnp
from jax import lax
from jax.experimental import pallas as pl
from jax.experimental.pallas import tpu as pltpu
from jax.experimental.pallas import tpu_sc as plsc

B = 16
NTAB = 3
H = 224
W = 224
D = 16
CH = 4
NCHUNK = H // CH
N = CH * W
NWORKERS = 32
ITEMS_TAB = B * NCHUNK
ITEMS_PER_W = ITEMS_TAB // NWORKERS
NGRP = N // 128

HW = H * W
TBLK = 25088
NBLK = HW // TBLK


def _gather_body(
    data_ref, ta_ref, tw_ref, tn_ref, out_ref, idx_v, rows_v, si, sw, sg
):
    nc = 2
    wid = lax.axis_index("s") * nc + lax.axis_index("c")

    def bc(t):
        q = wid * ITEMS_PER_W + t
        b = q // NCHUNK
        return b, q - b * NCHUNK

    for tab, table_ref in enumerate((ta_ref, tw_ref, tn_ref)):
        b0, c0 = bc(0)
        pltpu.async_copy(data_ref.at[b0, tab, c0], idx_v.at[0], si)

        def item_body(t, carry, tab=tab, table_ref=table_ref):
            p = lax.rem(t, 2)
            b, c = bc(t)

            pltpu.make_async_copy(data_ref.at[b, tab, c], idx_v.at[p], si).wait()

            @pl.when(t + 1 < ITEMS_PER_W)
            def _():
                bn, cn = bc(t + 1)
                pltpu.async_copy(data_ref.at[bn, tab, cn], idx_v.at[1 - p], si)

            @pl.when(t >= 2)
            def _():
                bo, co = bc(t - 2)
                pltpu.make_async_copy(
                    rows_v.at[p], out_ref.at[bo, tab, co], sw
                ).wait()

            copies = []
            for j in range(NGRP):
                copies.append(
                    pltpu.async_copy(
                        table_ref.at[idx_v.at[p, j]],
                        rows_v.at[p, pl.ds(j * 128, 128)],
                        sg,
                    )
                )
            for cp in copies:
                cp.wait()

            pltpu.async_copy(rows_v.at[p], out_ref.at[b, tab, c], sw)
            return carry

        lax.fori_loop(0, ITEMS_PER_W, item_body, 0)

        for t in (ITEMS_PER_W - 2, ITEMS_PER_W - 1):
            p = t % 2
            bt, ct = bc(t)
            pltpu.make_async_copy(rows_v.at[p], out_ref.at[bt, tab, ct], sw).wait()


def _transpose_body(x_ref, o_ref):
    x = x_ref[0]
    o_ref[0] = x.T


def kernel(data, W_areas, W_ways, W_nodes):
    data_r = data.astype(jnp.int32).reshape(B, NTAB, NCHUNK, NGRP, 128)

    mesh = plsc.VectorSubcoreMesh(core_axis_name="c", subcore_axis_name="s")
    gather = pl.kernel(
        _gather_body,
        out_type=jax.ShapeDtypeStruct((B, NTAB, NCHUNK, N, D), jnp.float32),
        mesh=mesh,
        compiler_params=pltpu.CompilerParams(use_tc_tiling_on_sc=False),
        scratch_types=[
            pltpu.VMEM((2, NGRP, 128), jnp.int32),
            pltpu.VMEM((2, N, D), jnp.float32),
            pltpu.SemaphoreType.DMA,
            pltpu.SemaphoreType.DMA,
            pltpu.SemaphoreType.DMA,
        ],
    )
    nat = gather(data_r, W_areas, W_ways, W_nodes)

    nat3 = nat.reshape(B * NTAB, HW, D)
    out3 = pl.pallas_call(
        _transpose_body,
        grid=(B * NTAB, NBLK),
        in_specs=[pl.BlockSpec((1, TBLK, D), lambda j, k: (j, k, 0))],
        out_specs=pl.BlockSpec((1, D, TBLK), lambda j, k: (j, 0, k)),
        out_shape=jax.ShapeDtypeStruct((B * NTAB, D, HW), jnp.float32),
    )(nat3)
    return out3.reshape(B, NTAB * D, H, W)
